# Initial kernel scaffold; baseline (speedup 1.0000x reference)
import jax
import jax.numpy as jnp
from jax.experimental import pallas as pl


def kernel(x, W_out, b_out, dictionary):
    raise NotImplementedError("write your pallas kernel here")


# speedup vs baseline: 3.8785x; 3.8785x over previous
import jax
import jax.numpy as jnp
from jax.experimental import pallas as pl
from jax.experimental.pallas import tpu as pltpu


def _fused_body(x_ref, w_ref, b_ref, d_ref, ids_ref, score_ref, quant_ref, loss_ref):
    DD = quant_ref.shape[-1]
    xb = x_ref[...]
    cont = jax.lax.dot_general(
        xb, w_ref[...], (((1,), (1,)), ((), ())),
        preferred_element_type=jnp.float32) + b_ref[...]
    logits = jax.lax.dot_general(
        cont, d_ref[:, :DD], (((1,), (1,)), ((), ())),
        preferred_element_type=jnp.float32)
    m = jnp.max(logits, axis=-1, keepdims=True)
    ids_ref[...] = jnp.argmax(logits, axis=-1).astype(jnp.int32)
    score_ref[...] = jnp.exp(logits - m)
    e = score_ref[...]
    q_aug = jax.lax.dot_general(
        e, d_ref[...], (((1,), (0,)), ((), ())),
        preferred_element_type=jnp.float32)
    rs = 1.0 / q_aug[:, DD:DD + 1]
    q = q_aug[:, :DD] * rs
    quant_ref[...] = q
    score_ref[...] = e * rs
    loss_ref[...] = jnp.full((1, 1, 1), jnp.sum((q - cont) ** 2), jnp.float32)


def kernel(x, W_out, b_out, dictionary):
    B, T, OUT_DIM = x.shape
    K = W_out.shape[0]
    V, DD = dictionary.shape
    N = B * T
    TB = min(256, N)
    nb = N // TB

    xf = x.reshape(N, OUT_DIM)
    b2 = b_out.reshape(1, K)
    d_aug = jnp.concatenate(
        [dictionary, jnp.ones((V, 128), jnp.float32)], axis=1)

    ids, score, quant, loss_sum = pl.pallas_call(
        _fused_body,
        grid=(nb,),
        compiler_params=pltpu.CompilerParams(
            vmem_limit_bytes=120 * 1024 * 1024,
            dimension_semantics=("parallel",),
        ),
        in_specs=[
            pl.BlockSpec((TB, OUT_DIM), lambda i: (i, 0)),
            pl.BlockSpec((K, OUT_DIM), lambda i: (0, 0)),
            pl.BlockSpec((1, K), lambda i: (0, 0)),
            pl.BlockSpec((V, DD + 128), lambda i: (0, 0)),
        ],
        out_specs=[
            pl.BlockSpec((TB,), lambda i: (i,)),
            pl.BlockSpec((TB, V), lambda i: (i, 0)),
            pl.BlockSpec((TB, DD), lambda i: (i, 0)),
            pl.BlockSpec((1, 1, 1), lambda i: (i, 0, 0)),
        ],
        out_shape=[
            jax.ShapeDtypeStruct((N,), jnp.int32),
            jax.ShapeDtypeStruct((N, V), jnp.float32),
            jax.ShapeDtypeStruct((N, DD), jnp.float32),
            jax.ShapeDtypeStruct((nb, 1, 1), jnp.float32),
        ],
    )(xf, W_out, b2, d_aug)

    ids = ids.reshape(B, T)
    score = score.reshape(B, T, V)
    quant = quant.reshape(B, T, DD)
    loss = jnp.sum(loss_sum) / jnp.float32(N * DD)
    return ids, score, quant, loss


# speedup vs baseline: 4.0084x; 1.0335x over previous
import jax
import jax.numpy as jnp
from jax.experimental import pallas as pl
from jax.experimental.pallas import tpu as pltpu


def _fused_body(x_ref, w_ref, b_ref, d_ref, o_ref, ids_ref, score_ref, quant_ref, loss_ref):
    DD = quant_ref.shape[-1]
    xb = x_ref[...]
    cont = jax.lax.dot_general(
        xb, w_ref[...], (((1,), (1,)), ((), ())),
        preferred_element_type=jnp.float32) + b_ref[...]
    logits = jax.lax.dot_general(
        cont, d_ref[...], (((1,), (1,)), ((), ())),
        preferred_element_type=jnp.float32)
    m = jnp.max(logits, axis=-1, keepdims=True)
    ids_ref[...] = jnp.argmax(logits, axis=-1).astype(jnp.int32)
    score_ref[...] = jnp.exp(logits - m)
    e = score_ref[...]
    qu = jax.lax.dot_general(
        e, d_ref[...], (((1,), (0,)), ((), ())),
        preferred_element_type=jnp.float32)
    s = jax.lax.dot_general(
        e, o_ref[...], (((1,), (0,)), ((), ())),
        preferred_element_type=jnp.float32)
    rs = 1.0 / s[:, :1]
    q = qu * rs
    quant_ref[...] = q
    score_ref[...] = e * rs
    loss_ref[...] = jnp.full((1, 1, 1), jnp.sum((q - cont) ** 2), jnp.float32)


def kernel(x, W_out, b_out, dictionary):
    B, T, OUT_DIM = x.shape
    K = W_out.shape[0]
    V, DD = dictionary.shape
    N = B * T
    TB = min(256, N)
    nb = N // TB

    xf = x.reshape(N, OUT_DIM)
    b2 = b_out.reshape(1, K)
    ones = jnp.ones((V, 8), jnp.float32)

    ids, score, quant, loss_sum = pl.pallas_call(
        _fused_body,
        grid=(nb,),
        compiler_params=pltpu.CompilerParams(
            vmem_limit_bytes=120 * 1024 * 1024,
            dimension_semantics=("parallel",),
        ),
        in_specs=[
            pl.BlockSpec((TB, OUT_DIM), lambda i: (i, 0)),
            pl.BlockSpec((K, OUT_DIM), lambda i: (0, 0)),
            pl.BlockSpec((1, K), lambda i: (0, 0)),
            pl.BlockSpec((V, DD), lambda i: (0, 0)),
            pl.BlockSpec((V, 8), lambda i: (0, 0)),
        ],
        out_specs=[
            pl.BlockSpec((TB,), lambda i: (i,)),
            pl.BlockSpec((TB, V), lambda i: (i, 0)),
            pl.BlockSpec((TB, DD), lambda i: (i, 0)),
            pl.BlockSpec((1, 1, 1), lambda i: (i, 0, 0)),
        ],
        out_shape=[
            jax.ShapeDtypeStruct((N,), jnp.int32),
            jax.ShapeDtypeStruct((N, V), jnp.float32),
            jax.ShapeDtypeStruct((N, DD), jnp.float32),
            jax.ShapeDtypeStruct((nb, 1, 1), jnp.float32),
        ],
    )(xf, W_out, b2, dictionary, ones)

    ids = ids.reshape(B, T)
    score = score.reshape(B, T, V)
    quant = quant.reshape(B, T, DD)
    loss = jnp.sum(loss_sum) / jnp.float32(N * DD)
    return ids, score, quant, loss
